# Initial kernel scaffold; baseline (speedup 1.0000x reference)
import jax
import jax.numpy as jnp
from jax.experimental import pallas as pl


def kernel(x, pos, batch, x_skip, pos_skip, batch_skip, W1, b1, g1, be1, W2, b2, g2, be2):
    raise NotImplementedError("write your pallas kernel here")


# speedup vs baseline: 10.9021x; 10.9021x over previous
import functools

import jax
import jax.numpy as jnp
from jax import lax
from jax.experimental import pallas as pl
from jax.experimental.pallas import tpu as pltpu
from jax.experimental.pallas import tpu_sc as plsc

N_COARSE = 2048
N_FINE = 8192
C_IN = 256
C_SKIP = 128
K = 3
EPS = 1e-5

QB = 256
NBLK = N_FINE // QB

NC = 2
NS = 16
NW = NC * NS
QPW = N_FINE // NW
SUB = 128


def _knn_body(ps_ref, bs_ref, pt_ref, bc_ref, idx_ref, w_ref):
    ps = ps_ref[...]
    pt = pt_ref[...]
    q2 = jnp.sum(ps * ps, axis=1, keepdims=True)
    c2 = jnp.sum(pt * pt, axis=0, keepdims=True)
    cross = jnp.dot(ps, pt, preferred_element_type=jnp.float32)
    d = q2 + c2 - 2.0 * cross
    mask = bs_ref[...] != bc_ref[...]
    d = jnp.where(mask, jnp.float32(1e10), d)

    iota = lax.broadcasted_iota(jnp.int32, (QB, N_COARSE), 1)
    idxs = []
    ws = []
    for _ in range(K):
        m = jnp.min(d, axis=1, keepdims=True)
        sel = d == m
        i = jnp.min(jnp.where(sel, iota, N_COARSE), axis=1, keepdims=True)
        idxs.append(i)
        ws.append(1.0 / jnp.maximum(m, jnp.float32(1e-16)))
        d = jnp.where(iota == i, jnp.float32(jnp.inf), d)
    wsum = ws[0] + ws[1] + ws[2]
    idx_ref[...] = jnp.concatenate(idxs, axis=1)
    w_ref[...] = jnp.concatenate([w / wsum for w in ws], axis=1)


def _knn_topk(pos_skip, batch_skip_f, pos_t, batch_f):
    return pl.pallas_call(
        _knn_body,
        grid=(NBLK,),
        in_specs=[
            pl.BlockSpec((QB, 3), lambda i: (i, 0)),
            pl.BlockSpec((QB, 1), lambda i: (i, 0)),
            pl.BlockSpec((3, N_COARSE), lambda i: (0, 0)),
            pl.BlockSpec((1, N_COARSE), lambda i: (0, 0)),
        ],
        out_specs=[
            pl.BlockSpec((QB, K), lambda i: (i, 0)),
            pl.BlockSpec((QB, K), lambda i: (i, 0)),
        ],
        out_shape=[
            jax.ShapeDtypeStruct((N_FINE, K), jnp.int32),
            jax.ShapeDtypeStruct((N_FINE, K), jnp.float32),
        ],
    )(pos_skip, batch_skip_f, pos_t, batch_f)


def _sc_gather_body(x_hbm, idx_hbm, out_hbm, idx_v, rows_v, sem):
    wid = lax.axis_index("s") * NC + lax.axis_index("c")
    for k in range(K):
        for s in range(QPW // SUB):
            off = k * N_FINE + wid * QPW + s * SUB
            pltpu.sync_copy(idx_hbm.at[pl.ds(off, SUB)], idx_v)
            pltpu.async_copy(x_hbm.at[idx_v], rows_v, sem).wait()
            pltpu.sync_copy(rows_v, out_hbm.at[pl.ds(off, SUB)])


@functools.cache
def _sc_gather_kernel():
    return pl.kernel(
        _sc_gather_body,
        out_type=jax.ShapeDtypeStruct((K * N_FINE, C_IN), jnp.float32),
        mesh=plsc.VectorSubcoreMesh(core_axis_name="c", subcore_axis_name="s",
                                    num_cores=NC, num_subcores=NS),
        scratch_types=[
            pltpu.VMEM((SUB,), jnp.int32),
            pltpu.VMEM((SUB, C_IN), jnp.float32),
            pltpu.SemaphoreType.DMA,
        ],
    )


def _sc_gather(x, idx_flat):
    return _sc_gather_kernel()(x, idx_flat)


def _m1_body(f0_ref, f1_ref, f2_ref, w_ref, xs_ref, W1_ref, b1_ref,
             h_ref, s_ref, ss_ref):
    w = w_ref[...]
    y = (w[:, 0:1] * f0_ref[...] + w[:, 1:2] * f1_ref[...]
         + w[:, 2:3] * f2_ref[...])
    h = (jnp.dot(y, W1_ref[0:C_IN, :], preferred_element_type=jnp.float32)
         + jnp.dot(xs_ref[...], W1_ref[C_IN:C_IN + C_SKIP, :],
                   preferred_element_type=jnp.float32)
         + b1_ref[...])

    @pl.when(pl.program_id(0) == 0)
    def _init():
        s_ref[...] = jnp.zeros_like(s_ref)
        ss_ref[...] = jnp.zeros_like(ss_ref)

    h_ref[...] = h
    s_ref[...] += jnp.sum(h, axis=0, keepdims=True)
    ss_ref[...] += jnp.sum(h * h, axis=0, keepdims=True)


def _m1(feats, w, x_skip, W1, b1):
    return pl.pallas_call(
        _m1_body,
        grid=(NBLK,),
        in_specs=[
            pl.BlockSpec((QB, C_IN), lambda i: (i, 0)),
            pl.BlockSpec((QB, C_IN), lambda i: (NBLK + i, 0)),
            pl.BlockSpec((QB, C_IN), lambda i: (2 * NBLK + i, 0)),
            pl.BlockSpec((QB, K), lambda i: (i, 0)),
            pl.BlockSpec((QB, C_SKIP), lambda i: (i, 0)),
            pl.BlockSpec((C_IN + C_SKIP, 256), lambda i: (0, 0)),
            pl.BlockSpec((1, 256), lambda i: (0, 0)),
        ],
        out_specs=[
            pl.BlockSpec((QB, 256), lambda i: (i, 0)),
            pl.BlockSpec((1, 256), lambda i: (0, 0)),
            pl.BlockSpec((1, 256), lambda i: (0, 0)),
        ],
        out_shape=[
            jax.ShapeDtypeStruct((N_FINE, 256), jnp.float32),
            jax.ShapeDtypeStruct((1, 256), jnp.float32),
            jax.ShapeDtypeStruct((1, 256), jnp.float32),
        ],
    )(feats, feats, feats, w, x_skip, W1, b1)


def _bn_silu(h, s, ss, g, be):
    mu = s * (1.0 / N_FINE)
    var = ss * (1.0 / N_FINE) - mu * mu
    hn = (h - mu) * lax.rsqrt(var + EPS) * g + be
    return hn * jax.nn.sigmoid(hn)


def _m2_body(h1_ref, s1_ref, ss1_ref, g1_ref, be1_ref, W2_ref, b2_ref,
             h_ref, s_ref, ss_ref):
    a = _bn_silu(h1_ref[...], s1_ref[...], ss1_ref[...], g1_ref[...],
                 be1_ref[...])
    h = (jnp.dot(a, W2_ref[...], preferred_element_type=jnp.float32)
         + b2_ref[...])

    @pl.when(pl.program_id(0) == 0)
    def _init():
        s_ref[...] = jnp.zeros_like(s_ref)
        ss_ref[...] = jnp.zeros_like(ss_ref)

    h_ref[...] = h
    s_ref[...] += jnp.sum(h, axis=0, keepdims=True)
    ss_ref[...] += jnp.sum(h * h, axis=0, keepdims=True)


def _m2(h1, s1, ss1, g1, be1, W2, b2):
    return pl.pallas_call(
        _m2_body,
        grid=(NBLK,),
        in_specs=[
            pl.BlockSpec((QB, 256), lambda i: (i, 0)),
            pl.BlockSpec((1, 256), lambda i: (0, 0)),
            pl.BlockSpec((1, 256), lambda i: (0, 0)),
            pl.BlockSpec((1, 256), lambda i: (0, 0)),
            pl.BlockSpec((1, 256), lambda i: (0, 0)),
            pl.BlockSpec((256, 256), lambda i: (0, 0)),
            pl.BlockSpec((1, 256), lambda i: (0, 0)),
        ],
        out_specs=[
            pl.BlockSpec((QB, 256), lambda i: (i, 0)),
            pl.BlockSpec((1, 256), lambda i: (0, 0)),
            pl.BlockSpec((1, 256), lambda i: (0, 0)),
        ],
        out_shape=[
            jax.ShapeDtypeStruct((N_FINE, 256), jnp.float32),
            jax.ShapeDtypeStruct((1, 256), jnp.float32),
            jax.ShapeDtypeStruct((1, 256), jnp.float32),
        ],
    )(h1, s1, ss1, g1, be1, W2, b2)


def _m3_body(h2_ref, s2_ref, ss2_ref, g2_ref, be2_ref, out_ref):
    out_ref[...] = _bn_silu(h2_ref[...], s2_ref[...], ss2_ref[...],
                            g2_ref[...], be2_ref[...])


def _m3(h2, s2, ss2, g2, be2):
    return pl.pallas_call(
        _m3_body,
        grid=(NBLK,),
        in_specs=[
            pl.BlockSpec((QB, 256), lambda i: (i, 0)),
            pl.BlockSpec((1, 256), lambda i: (0, 0)),
            pl.BlockSpec((1, 256), lambda i: (0, 0)),
            pl.BlockSpec((1, 256), lambda i: (0, 0)),
            pl.BlockSpec((1, 256), lambda i: (0, 0)),
        ],
        out_specs=pl.BlockSpec((QB, 256), lambda i: (i, 0)),
        out_shape=jax.ShapeDtypeStruct((N_FINE, 256), jnp.float32),
    )(h2, s2, ss2, g2, be2)


def kernel(x, pos, batch, x_skip, pos_skip, batch_skip,
           W1, b1, g1, be1, W2, b2, g2, be2):
    bs_f = batch_skip.astype(jnp.float32).reshape(N_FINE, 1)
    bc_f = batch.astype(jnp.float32).reshape(1, N_COARSE)
    pos_t = pos.T

    idx, w = _knn_topk(pos_skip, bs_f, pos_t, bc_f)
    idx_flat = idx.T.reshape(K * N_FINE)

    feats = _sc_gather(x, idx_flat)

    r = lambda v: v.reshape(1, 256)
    h1, s1, ss1 = _m1(feats, w, x_skip, W1, r(b1))
    h2, s2, ss2 = _m2(h1, s1, ss1, r(g1), r(be1), W2, r(b2))
    h = _m3(h2, s2, ss2, r(g2), r(be2))
    return (h, pos_skip, batch_skip)
